# baseline (speedup 1.0000x reference)
import jax
import jax.numpy as jnp
from jax import lax
from jax.experimental import pallas as pl
from jax.experimental.pallas import tpu as pltpu
from jax.experimental.pallas import tpu_sc as plsc

B, T, D = 16, 2048, 1024
SPLIT = 8
ROW = D // SPLIT


def _make_kernel():
    mesh = plsc.VectorSubcoreMesh(core_axis_name="c", subcore_axis_name="s")

    def run(t1, t2, idx):
        @pl.kernel(
            mesh=mesh,
            out_type=jax.ShapeDtypeStruct((B, 2, SPLIT, ROW), jnp.float32),
            scratch_types=[
                pltpu.VMEM((SPLIT,), jnp.int32),
                pltpu.VMEM((SPLIT, ROW), jnp.float32),
                pltpu.SemaphoreType.DMA,
            ],
        )
        def k(t1_hbm, t2_hbm, idx_hbm, out_hbm, idx_v, row_v, sem):
            wid = lax.axis_index("s") * 2 + lax.axis_index("c")
            b = wid % B
            t = wid // B
            pltpu.sync_copy(idx_hbm.at[wid], idx_v)

            @pl.when(t == 0)
            def _():
                pltpu.async_copy(t1_hbm.at[idx_v], row_v, sem).wait()

            @pl.when(t == 1)
            def _():
                pltpu.async_copy(t2_hbm.at[idx_v], row_v, sem).wait()

            pltpu.sync_copy(row_v, out_hbm.at[b, t])

        return k(t1, t2, idx)

    return run


_run = _make_kernel()


def kernel(output_lstm1, output_lstm2, input_length, support_length):
    t1 = output_lstm1.reshape(B * T * SPLIT, ROW)
    t2 = output_lstm2.reshape(B * T * SPLIT, ROW)

    arange_b = jnp.arange(B, dtype=jnp.int32)
    sub = jnp.arange(SPLIT, dtype=jnp.int32)
    il = input_length.astype(jnp.int32) - 1
    sl = support_length.astype(jnp.int32) - 1
    base1 = (arange_b * T + il) * SPLIT
    base2 = (arange_b * T + sl) * SPLIT
    idx = jnp.concatenate(
        [base1[:, None] + sub[None, :], base2[:, None] + sub[None, :]], axis=0
    )

    out = _run(t1, t2, idx)
    return out.reshape(B, 2 * D)


# speedup vs baseline: 12.4572x; 12.4572x over previous
import jax
import jax.numpy as jnp
from jax import lax
from jax.experimental import pallas as pl
from jax.experimental.pallas import tpu as pltpu
from jax.experimental.pallas import tpu_sc as plsc

B, T, D = 16, 2048, 1024


def _make_kernel():
    mesh = plsc.VectorSubcoreMesh(core_axis_name="c", subcore_axis_name="s")

    @pl.kernel(
        mesh=mesh,
        out_type=jax.ShapeDtypeStruct((B, 2, D), jnp.float32),
        scratch_types=[
            pltpu.VMEM((B,), jnp.int32),
            pltpu.VMEM((B,), jnp.int32),
            pltpu.VMEM((B, D), jnp.float32),
            pltpu.SemaphoreType.DMA,
        ],
    )
    def k(t1_hbm, t2_hbm, len1_hbm, len2_hbm, out_hbm, len_v, idx_v, rows_v, sem):
        wid = lax.axis_index("s") * 2 + lax.axis_index("c")

        @pl.when(wid == 0)
        def _():
            pltpu.sync_copy(len1_hbm, len_v)
            idx_v[...] = jnp.arange(B, dtype=jnp.int32) * T + len_v[...] - 1
            pltpu.async_copy(t1_hbm.at[idx_v], rows_v, sem).wait()
            pltpu.sync_copy(rows_v, out_hbm.at[:, 0])

        @pl.when(wid == 1)
        def _():
            pltpu.sync_copy(len2_hbm, len_v)
            idx_v[...] = jnp.arange(B, dtype=jnp.int32) * T + len_v[...] - 1
            pltpu.async_copy(t2_hbm.at[idx_v], rows_v, sem).wait()
            pltpu.sync_copy(rows_v, out_hbm.at[:, 1])

    return k


_run = _make_kernel()


def kernel(output_lstm1, output_lstm2, input_length, support_length):
    t1 = output_lstm1.reshape(B * T, D)
    t2 = output_lstm2.reshape(B * T, D)
    len1 = input_length.astype(jnp.int32)
    len2 = support_length.astype(jnp.int32)
    out = _run(t1, t2, len1, len2)
    return out.reshape(B, 2 * D)


# speedup vs baseline: 13.0643x; 1.0487x over previous
import jax
import jax.numpy as jnp
from jax import lax
from jax.experimental import pallas as pl
from jax.experimental.pallas import tpu as pltpu
from jax.experimental.pallas import tpu_sc as plsc

B, T, D = 16, 2048, 1024


def _make_kernel():
    mesh = plsc.VectorSubcoreMesh(core_axis_name="c", subcore_axis_name="s", num_cores=1)

    @pl.kernel(
        mesh=mesh,
        out_type=jax.ShapeDtypeStruct((B, 2, D), jnp.float32),
        scratch_types=[
            pltpu.VMEM((B,), jnp.int32),
            pltpu.VMEM((B,), jnp.int32),
            pltpu.VMEM((B, D), jnp.float32),
            pltpu.SemaphoreType.DMA,
        ],
    )
    def k(t1_hbm, t2_hbm, len1_hbm, len2_hbm, out_hbm, len_v, idx_v, rows_v, sem):
        wid = lax.axis_index("s")

        @pl.when(wid == 0)
        def _():
            pltpu.sync_copy(len1_hbm, len_v)
            idx_v[...] = jnp.arange(B, dtype=jnp.int32) * T + len_v[...] - 1
            pltpu.async_copy(t1_hbm.at[idx_v], rows_v, sem).wait()
            pltpu.sync_copy(rows_v, out_hbm.at[:, 0])

        @pl.when(wid == 1)
        def _():
            pltpu.sync_copy(len2_hbm, len_v)
            idx_v[...] = jnp.arange(B, dtype=jnp.int32) * T + len_v[...] - 1
            pltpu.async_copy(t2_hbm.at[idx_v], rows_v, sem).wait()
            pltpu.sync_copy(rows_v, out_hbm.at[:, 1])

    return k


_run = _make_kernel()


def kernel(output_lstm1, output_lstm2, input_length, support_length):
    t1 = output_lstm1.reshape(B * T, D)
    t2 = output_lstm2.reshape(B * T, D)
    len1 = input_length.astype(jnp.int32)
    len2 = support_length.astype(jnp.int32)
    out = _run(t1, t2, len1, len2)
    return out.reshape(B, 2 * D)
